# Initial kernel scaffold; baseline (speedup 1.0000x reference)
import jax
import jax.numpy as jnp
from jax.experimental import pallas as pl


def kernel(indices, hashes, weight):
    raise NotImplementedError("write your pallas kernel here")


# speedup vs baseline: 4.4248x; 4.4248x over previous
import functools

import jax
import jax.numpy as jnp
from jax import lax
from jax.experimental import pallas as pl
from jax.experimental.pallas import tpu as pltpu
from jax.experimental.pallas import tpu_sc as plsc

D = 32
G = 128
LANES = 16


def kernel(indices, hashes, weight):
    B, L = indices.shape
    N = B * L
    info = plsc.get_sparse_core_info()
    NW = info.num_cores * info.num_subcores
    NS = info.num_subcores
    n_groups = N // (NW * G)

    idx3 = indices.reshape(NW, n_groups, G)
    hflat = hashes.reshape(-1)

    @functools.partial(
        pl.kernel,
        mesh=plsc.VectorSubcoreMesh(core_axis_name="c", subcore_axis_name="s"),
        compiler_params=pltpu.CompilerParams(use_tc_tiling_on_sc=False),
        out_type=jax.ShapeDtypeStruct((NW, n_groups, G, D), jnp.float32),
        scratch_types=[
            pltpu.VMEM((n_groups, G), jnp.int32),
            pltpu.VMEM((n_groups, G), jnp.int32),
            pltpu.VMEM((n_groups, G), jnp.int32),
            pltpu.VMEM((G,), jnp.int32),
            pltpu.VMEM((G,), jnp.int32),
            pltpu.VMEM((G, D), jnp.float32),
            pltpu.VMEM((G, D), jnp.float32),
            pltpu.SemaphoreType.DMA,
        ],
    )
    def sc_kernel(idx_hbm, hflat_hbm, w_hbm, out_hbm,
                  idx_v, h0i, h1i, h0v, h1v, e0, e1, sem):
        wid = lax.axis_index("c") * NS + lax.axis_index("s")
        pltpu.sync_copy(idx_hbm.at[wid], idx_v)

        def compute_hidx(j, carry):
            for t in range(G // LANES):
                v = idx_v[j, pl.ds(t * LANES, LANES)]
                v2 = v * 2
                h0i[j, pl.ds(t * LANES, LANES)] = v2
                h1i[j, pl.ds(t * LANES, LANES)] = v2 + 1
            return carry

        lax.fori_loop(0, n_groups, compute_hidx, 0)

        def body(j, carry):
            c0 = pltpu.async_copy(hflat_hbm.at[h0i.at[j]], h0v, sem)
            c1 = pltpu.async_copy(hflat_hbm.at[h1i.at[j]], h1v, sem)
            c0.wait()
            c1.wait()
            g0 = pltpu.async_copy(w_hbm.at[h0v], e0, sem)
            g1 = pltpu.async_copy(w_hbm.at[h1v], e1, sem)
            g0.wait()
            g1.wait()

            def add_row(r, inner):
                for t in range(D // LANES):
                    e0[r, pl.ds(t * LANES, LANES)] = (
                        e0[r, pl.ds(t * LANES, LANES)]
                        + e1[r, pl.ds(t * LANES, LANES)]
                    )
                return inner

            lax.fori_loop(0, G, add_row, 0)
            pltpu.sync_copy(e0, out_hbm.at[wid, j])
            return carry

        lax.fori_loop(0, n_groups, body, 0)

    out = sc_kernel(idx3, hflat, weight)
    return out.reshape(B, L, D)


# speedup vs baseline: 4.9905x; 1.1278x over previous
import functools

import jax
import jax.numpy as jnp
from jax import lax
from jax.experimental import pallas as pl
from jax.experimental.pallas import tpu as pltpu
from jax.experimental.pallas import tpu_sc as plsc

D = 32
G = 128
K = 5
LANES = 16


def kernel(indices, hashes, weight):
    B, L = indices.shape
    N = B * L
    info = plsc.get_sparse_core_info()
    NW = info.num_cores * info.num_subcores
    NS = info.num_subcores
    n_groups = N // (NW * G)
    n_rounds = n_groups // K

    idx3 = indices.reshape(NW, n_groups, G)
    hflat = hashes.reshape(-1)

    @functools.partial(
        pl.kernel,
        mesh=plsc.VectorSubcoreMesh(core_axis_name="c", subcore_axis_name="s"),
        compiler_params=pltpu.CompilerParams(use_tc_tiling_on_sc=False),
        out_type=jax.ShapeDtypeStruct((NW, n_rounds, K * G, D), jnp.float32),
        scratch_types=[
            pltpu.VMEM((n_groups, G), jnp.int32),
            pltpu.VMEM((n_groups, G), jnp.int32),
            pltpu.VMEM((n_groups, G), jnp.int32),
            pltpu.VMEM((2, K, G), jnp.int32),
            pltpu.VMEM((2, K, G), jnp.int32),
            pltpu.VMEM((K * G, D), jnp.float32),
            pltpu.SemaphoreType.DMA,
            pltpu.SemaphoreType.DMA,
        ],
    )
    def sc_kernel(idx_hbm, hflat_hbm, w_hbm, out_hbm,
                  idx_v, h0i, h1i, h0v, h1v, ebuf, sem_h, sem_e):
        wid = lax.axis_index("c") * NS + lax.axis_index("s")
        pltpu.sync_copy(idx_hbm.at[wid], idx_v)

        def compute_hidx(j, carry):
            for t in range(G // LANES):
                v = idx_v[j, pl.ds(t * LANES, LANES)]
                v2 = v * 2
                h0i[j, pl.ds(t * LANES, LANES)] = v2
                h1i[j, pl.ds(t * LANES, LANES)] = v2 + 1
            return carry

        lax.fori_loop(0, n_groups, compute_hidx, 0)

        def fire_hash(r, p):
            cps = []
            for g in range(K):
                j = r * K + g
                cps.append(pltpu.async_copy(
                    hflat_hbm.at[h0i.at[j]], h0v.at[p, g], sem_h))
                cps.append(pltpu.async_copy(
                    hflat_hbm.at[h1i.at[j]], h1v.at[p, g], sem_h))
            return cps

        def round_body(r, p):
            for _ in range(2 * K):
                pltpu.make_async_copy(
                    hflat_hbm.at[h0i.at[0]], h0v.at[0, 0], sem_h).wait()
            e_cps = []
            for g in range(K):
                e_cps.append(pltpu.async_copy(
                    w_hbm.at[h0v.at[p, g]],
                    ebuf.at[pl.ds(g * G, G)], sem_e))

            @pl.when(r + 1 < n_rounds)
            def _():
                fire_hash(r + 1, 1 - p)

            for cp in e_cps:
                cp.wait()
            a_cps = []
            for g in range(K):
                a_cps.append(pltpu.async_copy(
                    w_hbm.at[h1v.at[p, g]],
                    ebuf.at[pl.ds(g * G, G)], sem_e, add=True))
            for cp in a_cps:
                cp.wait()
            pltpu.sync_copy(ebuf, out_hbm.at[wid, r])

        fire_hash(0, 0)

        def pair_body(t, carry):
            round_body(2 * t, 0)
            round_body(2 * t + 1, 1)
            return carry

        lax.fori_loop(0, n_rounds // 2, pair_body, 0)

    out = sc_kernel(idx3, hflat, weight)
    return out.reshape(B, L, D)


# speedup vs baseline: 26.1127x; 5.2325x over previous
import functools

import jax
import jax.numpy as jnp
from jax import lax
from jax.experimental import pallas as pl
from jax.experimental.pallas import tpu as pltpu
from jax.experimental.pallas import tpu_sc as plsc

D = 32
G = 128
K = 5


def kernel(indices, hashes, weight):
    B, L = indices.shape
    N = B * L
    info = plsc.get_sparse_core_info()
    NW = info.num_cores * info.num_subcores
    NS = info.num_subcores
    n_groups = N // (NW * G)
    n_rounds = n_groups // K

    idx3 = indices.reshape(NW, n_groups, G)
    h0col = hashes[:, 0]
    h1col = hashes[:, 1]

    @functools.partial(
        pl.kernel,
        mesh=plsc.VectorSubcoreMesh(core_axis_name="c", subcore_axis_name="s"),
        compiler_params=pltpu.CompilerParams(use_tc_tiling_on_sc=False),
        out_type=jax.ShapeDtypeStruct((NW, n_rounds, K * G, D), jnp.float32),
        scratch_types=[
            pltpu.VMEM((n_groups, G), jnp.int32),
            pltpu.VMEM((2, K, G), jnp.int32),
            pltpu.VMEM((2, K, G), jnp.int32),
            pltpu.VMEM((K * G, D), jnp.float32),
            pltpu.SemaphoreType.DMA,
            pltpu.SemaphoreType.DMA,
        ],
    )
    def sc_kernel(idx_hbm, h0_hbm, h1_hbm, w_hbm, out_hbm,
                  idx_v, h0v, h1v, ebuf, sem_h, sem_e):
        wid = lax.axis_index("c") * NS + lax.axis_index("s")
        pltpu.sync_copy(idx_hbm.at[wid], idx_v)

        def fire_hash(r, p):
            for g in range(K):
                j = r * K + g
                pltpu.async_copy(h0_hbm.at[idx_v.at[j]], h0v.at[p, g], sem_h)
                pltpu.async_copy(h1_hbm.at[idx_v.at[j]], h1v.at[p, g], sem_h)

        def round_body(r, p):
            for _ in range(2 * K):
                pltpu.make_async_copy(
                    h0_hbm.at[idx_v.at[0]], h0v.at[0, 0], sem_h).wait()
            e_cps = []
            for g in range(K):
                e_cps.append(pltpu.async_copy(
                    w_hbm.at[h0v.at[p, g]],
                    ebuf.at[pl.ds(g * G, G)], sem_e))

            @pl.when(r + 1 < n_rounds)
            def _():
                fire_hash(r + 1, 1 - p)

            for cp in e_cps:
                cp.wait()
            a_cps = []
            for g in range(K):
                a_cps.append(pltpu.async_copy(
                    w_hbm.at[h1v.at[p, g]],
                    ebuf.at[pl.ds(g * G, G)], sem_e, add=True))
            for cp in a_cps:
                cp.wait()
            pltpu.sync_copy(ebuf, out_hbm.at[wid, r])

        fire_hash(0, 0)

        def pair_body(t, carry):
            round_body(2 * t, 0)
            round_body(2 * t + 1, 1)
            return carry

        lax.fori_loop(0, n_rounds // 2, pair_body, 0)

    out = sc_kernel(idx3, h0col, h1col, weight)
    return out.reshape(B, L, D)
